# baseline (speedup 1.0000x reference)
import functools

import jax
import jax.numpy as jnp
from jax import lax
from jax.experimental import pallas as pl
from jax.experimental.pallas import tpu as pltpu
from jax.experimental.pallas import tpu_sc as plsc

_NUM_CHARS = 100000
_HIDDEN = 16
_BATCH = 1024
_TM = 2048
_LANE_TILES = 782
_SLAB = 3200


@functools.cache
def _make_sc_gather():
    info = plsc.get_sparse_core_info()
    nc, ns, nl = info.num_cores, info.num_subcores, info.num_lanes
    nw = nc * ns
    mesh = plsc.VectorSubcoreMesh(core_axis_name="c", subcore_axis_name="s")

    @functools.partial(
        pl.kernel,
        mesh=mesh,
        out_type=jax.ShapeDtypeStruct((nw, _HIDDEN, _BATCH), jnp.float32),
        scratch_types=[
            pltpu.VMEM((_BATCH,), jnp.int32),
            pltpu.VMEM((_HIDDEN, _SLAB), jnp.float32),
            pltpu.VMEM((_HIDDEN, _BATCH), jnp.float32),
        ],
        compiler_params=pltpu.CompilerParams(needs_layout_passes=False),
    )
    def gather_k(idx_hbm, tablet_hbm, out_hbm, idx_v, slab_v, xt_v):
        wid = lax.axis_index("s") * nc + lax.axis_index("c")
        lo_tile = (_LANE_TILES * wid) // nw
        hi_tile = (_LANE_TILES * (wid + 1)) // nw
        own_lo = lo_tile * 128
        own_hi = hi_tile * 128
        pltpu.sync_copy(idx_hbm, idx_v)
        pltpu.sync_copy(
            tablet_hbm.at[:, pl.ds(pl.multiple_of(own_lo, 128), _SLAB)],
            slab_v,
        )
        iota = lax.iota(jnp.int32, nl)

        zeros = jnp.zeros((nl,), jnp.float32)

        def chunk(c, carry):
            v = idx_v[pl.ds(c * nl, nl)]
            m = (v >= own_lo) & (v < own_hi)
            loc = jnp.clip(v - own_lo, 0, _SLAB - 1)
            cols = c * nl + iota

            def hit():
                for h in range(_HIDDEN):
                    hv = jnp.full((nl,), h, jnp.int32)
                    g = plsc.load_gather(slab_v, [hv, loc])
                    plsc.store_scatter(
                        xt_v, [hv, cols], jnp.where(m, g, jnp.float32(0.0)))

            def miss():
                for h in range(_HIDDEN):
                    hv = jnp.full((nl,), h, jnp.int32)
                    plsc.store_scatter(xt_v, [hv, cols], zeros)

            lax.cond(jnp.any(m), hit, miss)
            return carry

        lax.fori_loop(0, _BATCH // nl, chunk, 0, unroll=2)
        pltpu.sync_copy(xt_v, out_hbm.at[wid])

    return gather_k


def _matmul_body(xs_ref, projt_ref, out_ref, xt_vmem):
    i = pl.program_id(0)

    @pl.when(i == 0)
    def _():
        xt_vmem[...] = jnp.sum(xs_ref[...], axis=0)

    out_ref[...] = lax.dot_general(
        projt_ref[...],
        xt_vmem[...],
        dimension_numbers=(((0,), (0,)), ((), ())),
        preferred_element_type=jnp.float32,
    )


def _tc_matmul(xs, projt):
    nw = xs.shape[0]
    return pl.pallas_call(
        _matmul_body,
        grid=(pl.cdiv(_NUM_CHARS, _TM),),
        in_specs=[
            pl.BlockSpec((nw, _HIDDEN, _BATCH), lambda i: (0, 0, 0)),
            pl.BlockSpec((_HIDDEN, _TM), lambda i: (0, i)),
        ],
        out_specs=pl.BlockSpec((_TM, _BATCH), lambda i: (i, 0)),
        out_shape=jax.ShapeDtypeStruct((_NUM_CHARS, _BATCH), jnp.float32),
        scratch_shapes=[pltpu.VMEM((_HIDDEN, _BATCH), jnp.float32)],
    )(xs, projt)


def kernel(indices, embedding, projection_matrix):
    xs = _make_sc_gather()(indices.astype(jnp.int32), embedding.T)
    return _tc_matmul(xs, projection_matrix.T)


# speedup vs baseline: 1.0068x; 1.0068x over previous
import functools

import jax
import jax.numpy as jnp
from jax import lax
from jax.experimental import pallas as pl
from jax.experimental.pallas import tpu as pltpu
from jax.experimental.pallas import tpu_sc as plsc

_NUM_CHARS = 100000
_HIDDEN = 16
_BATCH = 1024
_TM = 2048
_LANE_TILES = 782
_SLAB = 3200


@functools.cache
def _make_sc_gather():
    info = plsc.get_sparse_core_info()
    nc, ns, nl = info.num_cores, info.num_subcores, info.num_lanes
    nw = nc * ns
    mesh = plsc.VectorSubcoreMesh(core_axis_name="c", subcore_axis_name="s")

    @functools.partial(
        pl.kernel,
        mesh=mesh,
        out_type=jax.ShapeDtypeStruct((nw, _HIDDEN, _BATCH), jnp.float32),
        scratch_types=[
            pltpu.VMEM((_BATCH,), jnp.int32),
            pltpu.VMEM((_HIDDEN, _SLAB), jnp.float32),
            pltpu.VMEM((_HIDDEN, _BATCH), jnp.float32),
            pltpu.SemaphoreType.DMA,
            pltpu.SemaphoreType.DMA,
        ],
        compiler_params=pltpu.CompilerParams(needs_layout_passes=False),
    )
    def gather_k(idx_hbm, tablet_hbm, out_hbm, idx_v, slab_v, xt_v, s1, s2):
        wid = lax.axis_index("s") * nc + lax.axis_index("c")
        lo_tile = (_LANE_TILES * wid) // nw
        hi_tile = (_LANE_TILES * (wid + 1)) // nw
        own_lo = lo_tile * 128
        own_hi = hi_tile * 128
        ci = pltpu.async_copy(idx_hbm, idx_v, s1)
        cs = pltpu.async_copy(
            tablet_hbm.at[:, pl.ds(pl.multiple_of(own_lo, 128), _SLAB)],
            slab_v,
            s2,
        )
        ci.wait()
        cs.wait()
        iota = lax.iota(jnp.int32, nl)

        zeros = jnp.zeros((nl,), jnp.float32)

        def chunk(c, carry):
            v = idx_v[pl.ds(c * nl, nl)]
            m = (v >= own_lo) & (v < own_hi)
            loc = jnp.clip(v - own_lo, 0, _SLAB - 1)
            cols = c * nl + iota

            def hit():
                for h in range(_HIDDEN):
                    hv = jnp.full((nl,), h, jnp.int32)
                    g = plsc.load_gather(slab_v, [hv, loc])
                    plsc.store_scatter(
                        xt_v, [hv, cols], jnp.where(m, g, jnp.float32(0.0)))

            def miss():
                for h in range(_HIDDEN):
                    hv = jnp.full((nl,), h, jnp.int32)
                    plsc.store_scatter(xt_v, [hv, cols], zeros)

            lax.cond(jnp.any(m), hit, miss)
            return carry

        lax.fori_loop(0, _BATCH // nl, chunk, 0, unroll=False)
        pltpu.sync_copy(xt_v, out_hbm.at[wid])

    return gather_k


def _matmul_body(xs_ref, projt_ref, out_ref, xt_vmem):
    i = pl.program_id(0)

    @pl.when(i == 0)
    def _():
        xt_vmem[...] = jnp.sum(xs_ref[...], axis=0)

    out_ref[...] = lax.dot_general(
        projt_ref[...],
        xt_vmem[...],
        dimension_numbers=(((0,), (0,)), ((), ())),
        preferred_element_type=jnp.float32,
    )


def _tc_matmul(xs, projt):
    nw = xs.shape[0]
    return pl.pallas_call(
        _matmul_body,
        grid=(pl.cdiv(_NUM_CHARS, _TM),),
        in_specs=[
            pl.BlockSpec((nw, _HIDDEN, _BATCH), lambda i: (0, 0, 0)),
            pl.BlockSpec((_HIDDEN, _TM), lambda i: (0, i)),
        ],
        out_specs=pl.BlockSpec((_TM, _BATCH), lambda i: (i, 0)),
        out_shape=jax.ShapeDtypeStruct((_NUM_CHARS, _BATCH), jnp.float32),
        scratch_shapes=[pltpu.VMEM((_HIDDEN, _BATCH), jnp.float32)],
    )(xs, projt)


def kernel(indices, embedding, projection_matrix):
    xs = _make_sc_gather()(indices.astype(jnp.int32), embedding.T)
    return _tc_matmul(xs, projection_matrix.T)
